# baseline (speedup 1.0000x reference)
import functools

import jax
import jax.numpy as jnp
from jax import lax
from jax.experimental import pallas as pl
from jax.experimental.pallas import tpu as pltpu
from jax.experimental.pallas import tpu_sc as plsc

_VOCAB = 1000


@functools.lru_cache(maxsize=None)
def _make_sc_onehot(n_batch: int, n_pos: int, vocab: int):
    info = plsc.get_sparse_core_info()
    num_cores, num_subcores, lanes = (
        info.num_cores, info.num_subcores, info.num_lanes)
    planes_per_core = n_pos // num_cores
    assert planes_per_core * num_cores == n_pos
    assert vocab % 8 == 0 and n_batch % 128 == 0
    plane_words = vocab * n_batch
    rows_per_tile = n_batch // num_subcores
    n_vec = rows_per_tile // lanes
    tile_row_words = 8 * n_batch

    total_words = n_pos * plane_words
    n_workers = num_cores * num_subcores
    span_words = total_words // n_workers
    n_zstream = 13
    zbuf_words = span_words // n_zstream
    assert zbuf_words * n_zstream == span_words

    mesh = plsc.VectorSubcoreMesh(core_axis_name="c", subcore_axis_name="s")

    @functools.partial(
        pl.kernel,
        mesh=mesh,
        out_type=jax.ShapeDtypeStruct((total_words,), jnp.float32),
        scratch_types=[
            pltpu.VMEM((zbuf_words,), jnp.float32),
            pltpu.VMEM((planes_per_core * rows_per_tile,), jnp.int32),
            pltpu.VMEM((planes_per_core, rows_per_tile), jnp.int32),
            pltpu.VMEM((rows_per_tile,), jnp.float32),
            pltpu.SemaphoreType.DMA,
            pltpu.SemaphoreType.DMA,
        ],
        compiler_params=pltpu.CompilerParams(needs_layout_passes=False),
    )
    def onehot(xt_hbm, out_hbm, zbuf, xv_all, idx2, ones_v, zsem, ssem):
        cid = lax.axis_index("c")
        sid = lax.axis_index("s")
        wid = cid * num_subcores + sid

        zeros16 = jnp.zeros((lanes,), jnp.float32)
        ones16 = jnp.ones((lanes,), jnp.float32)
        lane = lax.iota(jnp.int32, lanes)

        for j in range(n_vec):
            ones_v[pl.ds(j * lanes, lanes)] = ones16

        pltpu.sync_copy(
            xt_hbm.at[pl.ds(wid * (planes_per_core * rows_per_tile),
                            planes_per_core * rows_per_tile)],
            xv_all)

        def fill_body(i, c):
            zbuf[pl.ds(i * lanes, lanes)] = zeros16
            return c

        lax.fori_loop(0, zbuf_words // lanes, fill_body, 0)

        zhandles = []
        for i in range(n_zstream):
            zhandles.append(pltpu.make_async_copy(
                zbuf,
                out_hbm.at[pl.ds(wid * span_words + i * zbuf_words,
                                 zbuf_words)],
                zsem))
            zhandles[-1].start()

        for plane in range(planes_per_core):
            for j in range(n_vec):
                xv = xv_all[pl.ds(plane * rows_per_tile + j * lanes, lanes)]
                b = sid * rows_per_tile + j * lanes + lane
                tv = xv >> 3
                rest = ((xv & 7) << 7) + ((b >> 7) << 10) + (b & 127)
                pos = ((cid * planes_per_core + plane) * plane_words
                       + tv * tile_row_words + rest)
                idx2[plane, pl.ds(j * lanes, lanes)] = pos

        for h in zhandles:
            h.wait()
        plsc.subcore_barrier()

        shandles = []
        for plane in range(planes_per_core):
            shandles.append(pltpu.make_async_copy(
                ones_v, out_hbm.at[idx2.at[plane]], ssem))
            shandles[-1].start()
        for h in shandles:
            h.wait()

    return onehot


def kernel(x):
    n0, n1 = x.shape
    xt = (x.astype(jnp.int32).T
          .reshape(2, n1 // 2, 16, n0 // 16)
          .transpose(0, 2, 1, 3)
          .reshape(-1))
    flat = _make_sc_onehot(n0, n1, _VOCAB)(xt)
    y = flat.reshape(n1, _VOCAB // 8, n0 // 128, 8, 128)
    return jnp.transpose(y, (2, 4, 0, 1, 3)).reshape(n0, n1, _VOCAB)


# speedup vs baseline: 1.0978x; 1.0978x over previous
import functools

import jax
import jax.numpy as jnp
from jax import lax
from jax.experimental import pallas as pl
from jax.experimental.pallas import tpu as pltpu
from jax.experimental.pallas import tpu_sc as plsc

_VOCAB = 1000
_HEAD_TILES = 62


@functools.lru_cache(maxsize=None)
def _make_sc_onehot(n_batch: int, n_pos: int, vocab: int):
    info = plsc.get_sparse_core_info()
    num_cores, num_subcores, lanes = (
        info.num_cores, info.num_subcores, info.num_lanes)
    planes_per_core = n_pos // num_cores
    assert planes_per_core * num_cores == n_pos
    assert vocab % 8 == 0 and n_batch % 128 == 0
    plane_words = vocab * n_batch
    rows_per_tile = n_batch // num_subcores
    n_vec = rows_per_tile // lanes
    tile_row_words = 8 * n_batch
    n_tile_rows = vocab // 8

    split = _HEAD_TILES
    head_words = split * tile_row_words
    tail_words = plane_words - head_words
    assert tail_words % (8 * num_subcores) == 0
    tail_slot = tail_words // num_subcores

    trash_base = head_words
    align = num_subcores * lanes
    buf_words = ((trash_base + rows_per_tile + align - 1) // align) * align
    slot_words = buf_words // num_subcores
    assert slot_words % lanes == 0 and slot_words <= tail_slot

    mesh = plsc.VectorSubcoreMesh(core_axis_name="c", subcore_axis_name="s")

    @functools.partial(
        pl.kernel,
        mesh=mesh,
        out_type=jax.ShapeDtypeStruct((n_pos * plane_words,), jnp.float32),
        scratch_types=[
            pltpu.VMEM_SHARED((buf_words,), jnp.float32),
            pltpu.VMEM_SHARED((buf_words,), jnp.float32),
            pltpu.VMEM((tail_slot,), jnp.float32),
            pltpu.VMEM((planes_per_core * rows_per_tile,), jnp.int32),
            pltpu.VMEM((2 * rows_per_tile,), jnp.int32),
            pltpu.VMEM((2 * rows_per_tile,), jnp.int32),
            pltpu.VMEM((2 * rows_per_tile,), jnp.float32),
            pltpu.VMEM((2 * rows_per_tile,), jnp.float32),
            pltpu.VMEM((planes_per_core, rows_per_tile), jnp.int32),
            pltpu.VMEM((rows_per_tile,), jnp.float32),
            pltpu.SemaphoreType.DMA,
            pltpu.SemaphoreType.DMA,
            pltpu.SemaphoreType.DMA,
            pltpu.SemaphoreType.DMA,
        ],
        compiler_params=pltpu.CompilerParams(needs_layout_passes=False),
    )
    def onehot(xt_hbm, out_hbm, buf_a, buf_b, zbuf, xv_all,
               idx_a, idx_b, val_a, val_b, idx2, ones_v,
               sem_a, sem_b, zsem, ssem):
        bufs = (buf_a, buf_b)
        idx_bufs = (idx_a, idx_b)
        val_bufs = (val_a, val_b)
        sems = (sem_a, sem_b)
        cid = lax.axis_index("c")
        sid = lax.axis_index("s")
        wid = cid * num_subcores + sid

        zeros16 = jnp.zeros((lanes,), jnp.float32)
        ones16 = jnp.ones((lanes,), jnp.float32)
        lane = lax.iota(jnp.int32, lanes)

        for j in range(n_vec):
            ones_v[pl.ds(j * lanes, lanes)] = ones16
            trash16 = trash_base + j * lanes + lane
            for vb, ib in ((val_a, idx_a), (val_b, idx_b)):
                vb[pl.ds(j * lanes, lanes)] = zeros16
                vb[pl.ds(rows_per_tile + j * lanes, lanes)] = ones16
                ib[pl.ds(rows_per_tile + j * lanes, lanes)] = trash16

        pltpu.sync_copy(
            xt_hbm.at[pl.ds(wid * (planes_per_core * rows_per_tile),
                            planes_per_core * rows_per_tile)],
            xv_all)

        def fill_body(i, c):
            zbuf[pl.ds(i * lanes, lanes)] = zeros16
            return c

        lax.fori_loop(0, tail_slot // lanes, fill_body, 0)

        zhandles = []
        for plane in range(planes_per_core):
            tail0 = ((cid * planes_per_core + plane) * plane_words
                     + head_words + sid * tail_slot)
            zhandles.append(pltpu.make_async_copy(
                zbuf, out_hbm.at[pl.ds(tail0, tail_slot)], zsem))
            zhandles[-1].start()

        slot0 = sid * slot_words
        pltpu.sync_copy(zbuf.at[pl.ds(0, slot_words)],
                        buf_a.at[pl.ds(slot0, slot_words)])
        plsc.subcore_barrier()

        handles = {}
        for k in range(planes_per_core):
            buf = bufs[k % 2]
            idxb = idx_bufs[k % 2]
            valb = val_bufs[k % 2]
            if k >= 2:
                @pl.when(sid == 0)
                def _(h=handles[k - 2]):
                    h.wait()
                plsc.subcore_barrier()
            for j in range(n_vec):
                xv = xv_all[pl.ds(k * rows_per_tile + j * lanes, lanes)]
                b = sid * rows_per_tile + j * lanes + lane
                tv = xv >> 3
                rest = ((xv & 7) << 7) + ((b >> 7) << 10) + (b & 127)
                in_plane = tv * tile_row_words + rest
                pos = jnp.where(tv < split, in_plane,
                                trash_base + j * lanes + lane)
                stale = idxb[pl.ds(rows_per_tile + j * lanes, lanes)]
                idxb[pl.ds(j * lanes, lanes)] = stale
                valb[pl.ds(j * lanes, lanes)] = jnp.where(
                    stale == pos, ones16, zeros16)
                idxb[pl.ds(rows_per_tile + j * lanes, lanes)] = pos
                idx2[k, pl.ds(j * lanes, lanes)] = (
                    (cid * planes_per_core + k) * plane_words + in_plane)
            pltpu.sync_copy(valb, buf.at[idxb])
            plsc.subcore_barrier()

            out_off = (cid * (planes_per_core * plane_words)
                       + k * plane_words)
            handles[k] = pltpu.make_async_copy(
                buf.at[pl.ds(0, head_words)],
                out_hbm.at[pl.ds(out_off, head_words)],
                sems[k % 2])

            @pl.when(sid == 0)
            def _(h=handles[k]):
                h.start()

            if k == 0:
                pltpu.sync_copy(zbuf.at[pl.ds(0, slot_words)],
                                buf_b.at[pl.ds(slot0, slot_words)])
                plsc.subcore_barrier()

        for h in zhandles:
            h.wait()
        plsc.subcore_barrier()
        shandles = []
        for plane in range(planes_per_core):
            shandles.append(pltpu.make_async_copy(
                ones_v, out_hbm.at[idx2.at[plane]], ssem))
            shandles[-1].start()
        for h in shandles:
            h.wait()

        @pl.when(sid == 0)
        def _():
            handles[planes_per_core - 2].wait()
            handles[planes_per_core - 1].wait()

    return onehot


def kernel(x):
    n0, n1 = x.shape
    xt = (x.astype(jnp.int32).T
          .reshape(2, n1 // 2, 16, n0 // 16)
          .transpose(0, 2, 1, 3)
          .reshape(-1))
    flat = _make_sc_onehot(n0, n1, _VOCAB)(xt)
    y = flat.reshape(n1, _VOCAB // 8, n0 // 128, 8, 128)
    return jnp.transpose(y, (2, 4, 0, 1, 3)).reshape(n0, n1, _VOCAB)


# speedup vs baseline: 1.1150x; 1.0156x over previous
import functools

import jax
import jax.numpy as jnp
from jax import lax
from jax.experimental import pallas as pl
from jax.experimental.pallas import tpu as pltpu
from jax.experimental.pallas import tpu_sc as plsc

_VOCAB = 1000


@functools.lru_cache(maxsize=None)
def _make_sc_onehot(n_batch: int, n_pos: int, vocab: int):
    info = plsc.get_sparse_core_info()
    num_cores, num_subcores, lanes = (
        info.num_cores, info.num_subcores, info.num_lanes)
    planes_per_core = n_pos // num_cores
    assert planes_per_core * num_cores == n_pos
    assert vocab % 8 == 0 and n_batch % 128 == 0
    plane_words = vocab * n_batch
    rows_per_tile = n_batch // num_subcores
    n_vec = rows_per_tile // lanes

    tile_row_words = 8 * n_batch
    n_tile_rows = vocab // 8
    split = n_tile_rows // 2
    chunk_words = (split * tile_row_words,
                   (n_tile_rows - split) * tile_row_words)
    trash_base = max(chunk_words)
    align = num_subcores * lanes
    buf_words = ((trash_base + rows_per_tile + align - 1) // align) * align
    slot_words = buf_words // num_subcores

    mesh = plsc.VectorSubcoreMesh(core_axis_name="c", subcore_axis_name="s")

    @functools.partial(
        pl.kernel,
        mesh=mesh,
        out_type=jax.ShapeDtypeStruct((n_pos * plane_words,), jnp.float32),
        scratch_types=[
            pltpu.VMEM_SHARED((buf_words,), jnp.float32),
            pltpu.VMEM_SHARED((buf_words,), jnp.float32),
            pltpu.VMEM((slot_words,), jnp.float32),
            pltpu.VMEM((planes_per_core * rows_per_tile,), jnp.int32),
            pltpu.VMEM((2 * rows_per_tile,), jnp.int32),
            pltpu.VMEM((2 * rows_per_tile,), jnp.int32),
            pltpu.VMEM((2 * rows_per_tile,), jnp.float32),
            pltpu.VMEM((2 * rows_per_tile,), jnp.float32),
            pltpu.SemaphoreType.DMA,
            pltpu.SemaphoreType.DMA,
        ],
        compiler_params=pltpu.CompilerParams(needs_layout_passes=False),
    )
    def onehot(xt_hbm, out_hbm, buf_a, buf_b, zbuf, xv_all,
               idx_a, idx_b, val_a, val_b, sem_a, sem_b):
        bufs = (buf_a, buf_b)
        idx_bufs = (idx_a, idx_b)
        val_bufs = (val_a, val_b)
        sems = (sem_a, sem_b)
        cid = lax.axis_index("c")
        sid = lax.axis_index("s")

        zeros16 = jnp.zeros((lanes,), jnp.float32)
        ones16 = jnp.ones((lanes,), jnp.float32)
        lane = lax.iota(jnp.int32, lanes)

        for j in range(n_vec):
            trash16 = trash_base + j * lanes + lane
            for vb, ib in ((val_a, idx_a), (val_b, idx_b)):
                vb[pl.ds(j * lanes, lanes)] = zeros16
                vb[pl.ds(rows_per_tile + j * lanes, lanes)] = ones16
                ib[pl.ds(rows_per_tile + j * lanes, lanes)] = trash16

        pltpu.sync_copy(
            xt_hbm.at[pl.ds((cid * num_subcores + sid)
                            * (planes_per_core * rows_per_tile),
                            planes_per_core * rows_per_tile)],
            xv_all)

        def fill_body(i, c):
            zbuf[pl.ds(i * lanes, lanes)] = zeros16
            return c

        lax.fori_loop(0, slot_words // lanes, fill_body, 0)

        slot0 = sid * slot_words
        pltpu.sync_copy(zbuf, buf_a.at[pl.ds(slot0, slot_words)])
        plsc.subcore_barrier()

        n_chunks = 2 * planes_per_core
        handles = {}
        for k in range(n_chunks):
            plane, part = k // 2, k % 2
            buf = bufs[k % 2]
            idxb = idx_bufs[k % 2]
            valb = val_bufs[k % 2]
            if k >= 2:
                @pl.when(sid == 0)
                def _(h=handles[k - 2]):
                    h.wait()
                plsc.subcore_barrier()
            for j in range(n_vec):
                xv = xv_all[pl.ds(plane * rows_per_tile + j * lanes, lanes)]
                b = sid * rows_per_tile + j * lanes + lane
                tv = xv >> 3
                rest = ((xv & 7) << 7) + ((b >> 7) << 10) + (b & 127)
                if part == 0:
                    pos = jnp.where(tv < split, tv * tile_row_words + rest,
                                    trash_base + j * lanes + lane)
                else:
                    pos = jnp.where(tv >= split,
                                    (tv - split) * tile_row_words + rest,
                                    trash_base + j * lanes + lane)
                stale = idxb[pl.ds(rows_per_tile + j * lanes, lanes)]
                idxb[pl.ds(j * lanes, lanes)] = stale
                valb[pl.ds(j * lanes, lanes)] = jnp.where(
                    stale == pos, ones16, zeros16)
                idxb[pl.ds(rows_per_tile + j * lanes, lanes)] = pos
            pltpu.sync_copy(valb, buf.at[idxb])
            plsc.subcore_barrier()

            out_off = (cid * (planes_per_core * plane_words)
                       + plane * plane_words + part * chunk_words[0])
            handles[k] = pltpu.make_async_copy(
                buf.at[pl.ds(0, chunk_words[part])],
                out_hbm.at[pl.ds(out_off, chunk_words[part])],
                sems[k % 2])

            @pl.when(sid == 0)
            def _(h=handles[k]):
                h.start()

            if k == 0:
                pltpu.sync_copy(zbuf, buf_b.at[pl.ds(slot0, slot_words)])
                plsc.subcore_barrier()

        @pl.when(sid == 0)
        def _():
            handles[n_chunks - 2].wait()
            handles[n_chunks - 1].wait()

    return onehot


def kernel(x):
    n0, n1 = x.shape
    xt = (x.astype(jnp.int32).T
          .reshape(2, n1 // 2, 16, n0 // 16)
          .transpose(0, 2, 1, 3)
          .reshape(-1))
    flat = _make_sc_onehot(n0, n1, _VOCAB)(xt)
    y = flat.reshape(n1, _VOCAB // 8, n0 // 128, 8, 128)
    return jnp.transpose(y, (2, 4, 0, 1, 3)).reshape(n0, n1, _VOCAB)
